# baseline (speedup 1.0000x reference)
import functools

import jax
import jax.numpy as jnp
from jax import lax
from jax.experimental import pallas as pl
from jax.experimental.pallas import tpu as pltpu
from jax.experimental.pallas import tpu_sc as plsc

DIM = 64
DPAD = 128
NROW = 1000000
B = 1024
L = 50
LP = 56
ALPHA = 0.2
NEG = -1e9

NC = 2
NS = 16
NW = NC * NS
NIDX = 2 * B * LP
BPW = NIDX // NW
CH = 1792
HALF = B * LP

CT = 4096


def _tpad_body(x_ref, o_ref):
    x = x_ref[...]
    eyep = (lax.broadcasted_iota(jnp.int32, (DIM, DPAD), 0)
            == lax.broadcasted_iota(jnp.int32, (DIM, DPAD), 1)
            ).astype(jnp.float32)
    o_ref[...] = lax.dot_general(x, eyep, (((0,), (0,)), ((), ())),
                                 preferred_element_type=jnp.float32
                                 ).astype(jnp.bfloat16)


def _tpad(embT):
    return pl.pallas_call(
        _tpad_body,
        grid=(pl.cdiv(NROW, CT),),
        in_specs=[pl.BlockSpec((DIM, CT), lambda i: (0, i))],
        out_specs=pl.BlockSpec((CT, DPAD), lambda i: (i, 0)),
        out_shape=jax.ShapeDtypeStruct((NROW, DPAD), jnp.bfloat16),
    )(embT)


def _gather_body(table, idx, out_h, out_i, idx_v, rows_v, sem):
    wid = lax.axis_index("s") * NC + lax.axis_index("c")
    base = wid * BPW
    pltpu.sync_copy(idx.at[pl.ds(base, BPW)], idx_v)
    for c in range(BPW // CH):
        pltpu.async_copy(table.at[idx_v.at[pl.ds(c * CH, CH)]], rows_v, sem).wait()
        off = base + c * CH

        @pl.when(wid < NW // 2)
        def _():
            pltpu.sync_copy(rows_v, out_h.at[pl.ds(off, CH)])

        @pl.when(wid >= NW // 2)
        def _():
            pltpu.sync_copy(rows_v, out_i.at[pl.ds(off - HALF, CH)])


@functools.cache
def _sc_gather():
    return pl.kernel(
        _gather_body,
        out_type=(
            jax.ShapeDtypeStruct((HALF, DPAD), jnp.bfloat16),
            jax.ShapeDtypeStruct((HALF, DPAD), jnp.bfloat16),
        ),
        mesh=plsc.VectorSubcoreMesh(
            core_axis_name="c", subcore_axis_name="s",
            num_cores=NC, num_subcores=NS,
        ),
        scratch_types=[
            pltpu.VMEM((BPW,), jnp.int32),
            pltpu.VMEM((CH, DPAD), jnp.bfloat16),
            pltpu.SemaphoreType.DMA,
        ],
        compiler_params=pltpu.CompilerParams(use_tc_tiling_on_sc=False),
    )


BB = 64


def _attn_body(h_ref, it_ref, ii_ref, ti_ref, hs_ref, mf_ref, a1_ref, a2_ref,
               out_ref):
    hs = hs_ref[...]
    mf = mf_ref[...][:, :L]
    a1 = a1_ref[...][0:1, :].reshape(1, 1, DPAD)
    a2 = a2_ref[...][0:1, :].reshape(1, 1, DPAD)

    h = h_ref[...][:, :L, :].astype(jnp.float32)
    h = h * (ii_ref[...][:, :L] != 0).astype(jnp.float32)[:, :, None]
    itm = it_ref[...][:, :L, :].astype(jnp.float32)
    itm = itm * ((ti_ref[...][:, :L] != 0).astype(jnp.float32) * mf)[:, :, None]
    sess = jnp.sum(itm, axis=1) / jnp.sum(mf, axis=1)[:, None]

    att_n = jnp.sum(h * a1, axis=-1)
    att_n = jnp.where(att_n >= 0, att_n, ALPHA * att_n)
    w = jnp.where(hs > 0, hs * att_n[:, :, None], NEG)
    w = w - jnp.max(w, axis=1, keepdims=True)
    w = jnp.exp(w)
    w = w / jnp.sum(w, axis=1, keepdims=True)
    edge = lax.dot_general(w, h, (((1,), (1,)), ((0,), (0,))),
                           preferred_element_type=jnp.float32)

    att_e = jnp.sum(edge * a2, axis=-1)
    att_e = jnp.where(att_e >= 0, att_e, ALPHA * att_e)
    w2 = jnp.where(hs > 0, hs * att_e[:, None, :], NEG)
    w2 = w2 - jnp.max(w2, axis=2, keepdims=True)
    w2 = jnp.exp(w2)
    w2 = w2 / jnp.sum(w2, axis=2, keepdims=True)
    out = lax.dot_general(w2, edge, (((2,), (1,)), ((0,), (0,))),
                          preferred_element_type=jnp.float32)
    out = out + sess[:, None, :]
    out_ref[...] = out[:, :, :DIM]


def _attention(h, itm, ii, ti, Hs, maskf, a1b, a2b):
    grid = (B // BB,)
    return pl.pallas_call(
        _attn_body,
        grid=grid,
        in_specs=[
            pl.BlockSpec((BB, LP, DPAD), lambda i: (i, 0, 0)),
            pl.BlockSpec((BB, LP, DPAD), lambda i: (i, 0, 0)),
            pl.BlockSpec((BB, LP), lambda i: (i, 0)),
            pl.BlockSpec((BB, LP), lambda i: (i, 0)),
            pl.BlockSpec((BB, L, L), lambda i: (i, 0, 0)),
            pl.BlockSpec((BB, LP), lambda i: (i, 0)),
            pl.BlockSpec((8, DPAD), lambda i: (0, 0)),
            pl.BlockSpec((8, DPAD), lambda i: (0, 0)),
        ],
        out_specs=pl.BlockSpec((BB, L, DIM), lambda i: (i, 0, 0)),
        out_shape=jax.ShapeDtypeStruct((B, L, DIM), jnp.float32),
    )(h, itm, ii, ti, Hs, maskf, a1b, a2b)


def kernel(inputs, Hs, mask_item, item, embedding, a1, a2):
    inputs = inputs.astype(jnp.int32)
    item = item.astype(jnp.int32)
    fb = (lax.broadcasted_iota(jnp.int32, (B, LP - L), 0) * 131
          + lax.broadcasted_iota(jnp.int32, (B, LP - L), 1) * 17) % NROW
    ii56 = jnp.concatenate([inputs, fb], axis=1)
    ti56 = jnp.concatenate([item, (fb + 7) % NROW], axis=1)
    idx = jnp.concatenate([ii56.reshape(-1), ti56.reshape(-1)])
    safe = jnp.maximum(idx - 1, 0)

    table = _tpad(embedding.T)
    h_rows, item_rows = _sc_gather()(table, safe)
    h3 = h_rows.reshape(B, LP, DPAD)
    it3 = item_rows.reshape(B, LP, DPAD)

    mf56 = jnp.pad(mask_item.astype(jnp.float32), ((0, 0), (0, LP - L)))
    a1b = jnp.pad(a1.reshape(1, DIM), ((0, 7), (0, DPAD - DIM)))
    a2b = jnp.pad(a2.reshape(1, DIM), ((0, 7), (0, DPAD - DIM)))
    h_local = _attention(h3, it3, ii56, ti56, Hs, mf56, a1b, a2b)

    item_embeddings = jnp.concatenate(
        [jnp.zeros((1, DIM), dtype=embedding.dtype), embedding], axis=0)
    return (h_local, item_embeddings)


# speedup vs baseline: 2.6925x; 2.6925x over previous
import functools

import jax
import jax.numpy as jnp
from jax import lax
from jax.experimental import pallas as pl
from jax.experimental.pallas import tpu as pltpu
from jax.experimental.pallas import tpu_sc as plsc

DIM = 64
DPAD = 128
NROW = 1000000
B = 1024
L = 50
LP = 56
ALPHA = 0.2
NEG = -1e9

NC = 2
NS = 16
NW = NC * NS
NIDX = 2 * B * LP
BPW = NIDX // NW
CH = 896
HALF = B * LP

CT = 8192


def _tpad_body(x_ref, o_ref):
    x = x_ref[...]
    eyep = (lax.broadcasted_iota(jnp.int32, (DIM, DPAD), 0)
            == lax.broadcasted_iota(jnp.int32, (DIM, DPAD), 1)
            ).astype(jnp.float32)
    o_ref[...] = lax.dot_general(x, eyep, (((0,), (0,)), ((), ())),
                                 preferred_element_type=jnp.float32)


def _tpad(embT):
    return pl.pallas_call(
        _tpad_body,
        grid=(pl.cdiv(NROW, CT),),
        in_specs=[pl.BlockSpec((DIM, CT), lambda i: (0, i))],
        out_specs=pl.BlockSpec((CT, DPAD), lambda i: (i, 0)),
        out_shape=jax.ShapeDtypeStruct((NROW, DPAD), jnp.float32),
    )(embT)


def _gather_body(table, idx, out_h, out_i, idx_v, rows_v, sem):
    wid = lax.axis_index("s") * NC + lax.axis_index("c")
    base = wid * BPW
    pltpu.sync_copy(idx.at[pl.ds(base, BPW)], idx_v)
    for c in range(BPW // CH):
        pltpu.async_copy(table.at[idx_v.at[pl.ds(c * CH, CH)]], rows_v, sem).wait()
        off = base + c * CH

        @pl.when(wid < NW // 2)
        def _():
            pltpu.sync_copy(rows_v, out_h.at[pl.ds(off, CH)])

        @pl.when(wid >= NW // 2)
        def _():
            pltpu.sync_copy(rows_v, out_i.at[pl.ds(off - HALF, CH)])


@functools.cache
def _sc_gather():
    return pl.kernel(
        _gather_body,
        out_type=(
            jax.ShapeDtypeStruct((HALF, DPAD), jnp.float32),
            jax.ShapeDtypeStruct((HALF, DPAD), jnp.float32),
        ),
        mesh=plsc.VectorSubcoreMesh(
            core_axis_name="c", subcore_axis_name="s",
            num_cores=NC, num_subcores=NS,
        ),
        scratch_types=[
            pltpu.VMEM((BPW,), jnp.int32),
            pltpu.VMEM((CH, DPAD), jnp.float32),
            pltpu.SemaphoreType.DMA,
        ],
        compiler_params=pltpu.CompilerParams(use_tc_tiling_on_sc=False),
    )


BB = 64


def _attn_body(h_ref, it_ref, ii_ref, ti_ref, hs_ref, mf_ref, a1_ref, a2_ref,
               out_ref):
    hst = hs_ref[...]
    mf = mf_ref[...][:, :L]
    a1 = a1_ref[...][0:1, :].reshape(1, 1, DPAD)
    a2 = a2_ref[...][0:1, :].reshape(1, 1, DPAD)

    h = h_ref[...][:, :L, :]
    h = h * (ii_ref[...][:, :L] != 0).astype(jnp.float32)[:, :, None]
    itm = it_ref[...][:, :L, :]
    itm = itm * ((ti_ref[...][:, :L] != 0).astype(jnp.float32) * mf)[:, :, None]
    sess = jnp.sum(itm, axis=1) / jnp.sum(mf, axis=1)[:, None]

    att_n = jnp.sum(h * a1, axis=-1)
    att_n = jnp.where(att_n >= 0, att_n, ALPHA * att_n)
    w = jnp.where(hst > 0, hst * att_n[:, None, :], NEG)
    w = w - jnp.max(w, axis=2, keepdims=True)
    w = jnp.exp(w)
    w = w / jnp.sum(w, axis=2, keepdims=True)
    edge = lax.dot_general(w, h, (((2,), (1,)), ((0,), (0,))),
                           preferred_element_type=jnp.float32)

    att_e = jnp.sum(edge * a2, axis=-1)
    att_e = jnp.where(att_e >= 0, att_e, ALPHA * att_e)
    w2 = jnp.where(hst > 0, hst * att_e[:, :, None], NEG)
    w2 = w2 - jnp.max(w2, axis=1, keepdims=True)
    w2 = jnp.exp(w2)
    w2 = w2 / jnp.sum(w2, axis=1, keepdims=True)
    out = lax.dot_general(w2, edge, (((1,), (1,)), ((0,), (0,))),
                          preferred_element_type=jnp.float32)
    out = out + sess[:, None, :]
    out_ref[...] = out[:, :, :DIM]


def _attention(h, itm, ii, ti, Hs, maskf, a1b, a2b):
    grid = (B // BB,)
    return pl.pallas_call(
        _attn_body,
        grid=grid,
        in_specs=[
            pl.BlockSpec((BB, LP, DPAD), lambda i: (i, 0, 0)),
            pl.BlockSpec((BB, LP, DPAD), lambda i: (i, 0, 0)),
            pl.BlockSpec((BB, LP), lambda i: (i, 0)),
            pl.BlockSpec((BB, LP), lambda i: (i, 0)),
            pl.BlockSpec((BB, L, L), lambda i: (i, 0, 0)),
            pl.BlockSpec((BB, LP), lambda i: (i, 0)),
            pl.BlockSpec((8, DPAD), lambda i: (0, 0)),
            pl.BlockSpec((8, DPAD), lambda i: (0, 0)),
        ],
        out_specs=pl.BlockSpec((BB, L, DIM), lambda i: (i, 0, 0)),
        out_shape=jax.ShapeDtypeStruct((B, L, DIM), jnp.float32),
    )(h, itm, ii, ti, Hs, maskf, a1b, a2b)


def kernel(inputs, Hs, mask_item, item, embedding, a1, a2):
    inputs = inputs.astype(jnp.int32)
    item = item.astype(jnp.int32)
    fb = (lax.broadcasted_iota(jnp.int32, (B, LP - L), 0) * 131
          + lax.broadcasted_iota(jnp.int32, (B, LP - L), 1) * 17) % NROW
    ii56 = jnp.concatenate([inputs, fb], axis=1)
    ti56 = jnp.concatenate([item, (fb + 7) % NROW], axis=1)
    idx = jnp.concatenate([ii56.reshape(-1), ti56.reshape(-1)])
    safe = jnp.maximum(idx - 1, 0)

    table = _tpad(embedding.T)
    h_rows, item_rows = _sc_gather()(table, safe)
    h3 = h_rows.reshape(B, LP, DPAD)
    it3 = item_rows.reshape(B, LP, DPAD)

    mf56 = jnp.pad(mask_item.astype(jnp.float32), ((0, 0), (0, LP - L)))
    a1b = jnp.pad(a1.reshape(1, DIM), ((0, 7), (0, DPAD - DIM)))
    a2b = jnp.pad(a2.reshape(1, DIM), ((0, 7), (0, DPAD - DIM)))
    h_local = _attention(h3, it3, ii56, ti56, jnp.swapaxes(Hs, 1, 2),
                         mf56, a1b, a2b)

    item_embeddings = jnp.concatenate(
        [jnp.zeros((1, DIM), dtype=embedding.dtype), embedding], axis=0)
    return (h_local, item_embeddings)


# speedup vs baseline: 3.1751x; 1.1792x over previous
import functools

import jax
import jax.numpy as jnp
from jax import lax
from jax.experimental import pallas as pl
from jax.experimental.pallas import tpu as pltpu
from jax.experimental.pallas import tpu_sc as plsc

DIM = 64
DPAD = 128
NROW = 1000000
B = 1024
L = 50
LP = 56
ALPHA = 0.2
NEG = -1e9

NC = 2
NS = 16
NW = NC * NS
NIDX = 2 * B * LP
BPW = NIDX // NW
CH = 896
HALF = B * LP

CT = 8192


def _build_body(x_ref, pt_ref, tab_ref, emb_ref):
    i = pl.program_id(0)
    x = x_ref[...]
    eyep = (lax.broadcasted_iota(jnp.int32, (DIM, DPAD), 0)
            == lax.broadcasted_iota(jnp.int32, (DIM, DPAD), 1)
            ).astype(jnp.float32)
    tab_ref[...] = lax.dot_general(x, eyep, (((0,), (0,)), ((), ())),
                                   preferred_element_type=jnp.float32)
    pt = pt_ref[...]
    first = jnp.where(i == 0, 0.0, 1.0) * pt[:, DPAD - 1:DPAD]
    emb_ref[...] = jnp.concatenate([first, x[:, :CT - 1]], axis=1)


def _build(embT):
    return pl.pallas_call(
        _build_body,
        grid=(pl.cdiv(NROW + 1, CT),),
        in_specs=[
            pl.BlockSpec((DIM, CT), lambda i: (0, i)),
            pl.BlockSpec((DIM, DPAD),
                         lambda i: (0, jnp.maximum(i * (CT // DPAD) - 1, 0))),
        ],
        out_specs=[
            pl.BlockSpec((CT, DPAD), lambda i: (i, 0)),
            pl.BlockSpec((DIM, CT), lambda i: (0, i)),
        ],
        out_shape=(
            jax.ShapeDtypeStruct((NROW, DPAD), jnp.float32),
            jax.ShapeDtypeStruct((DIM, NROW + 1), jnp.float32),
        ),
    )(embT, embT)


def _gather_body(table, idx, out_h, out_i, idx_v, rows_v, sem):
    wid = lax.axis_index("s") * NC + lax.axis_index("c")
    base = wid * BPW
    pltpu.sync_copy(idx.at[pl.ds(base, BPW)], idx_v)
    for c in range(BPW // CH):
        pltpu.async_copy(table.at[idx_v.at[pl.ds(c * CH, CH)]], rows_v, sem).wait()
        off = base + c * CH

        @pl.when(wid < NW // 2)
        def _():
            pltpu.sync_copy(rows_v, out_h.at[pl.ds(off, CH)])

        @pl.when(wid >= NW // 2)
        def _():
            pltpu.sync_copy(rows_v, out_i.at[pl.ds(off - HALF, CH)])


@functools.cache
def _sc_gather():
    return pl.kernel(
        _gather_body,
        out_type=(
            jax.ShapeDtypeStruct((HALF, DPAD), jnp.float32),
            jax.ShapeDtypeStruct((HALF, DPAD), jnp.float32),
        ),
        mesh=plsc.VectorSubcoreMesh(
            core_axis_name="c", subcore_axis_name="s",
            num_cores=NC, num_subcores=NS,
        ),
        scratch_types=[
            pltpu.VMEM((BPW,), jnp.int32),
            pltpu.VMEM((CH, DPAD), jnp.float32),
            pltpu.SemaphoreType.DMA,
        ],
        compiler_params=pltpu.CompilerParams(use_tc_tiling_on_sc=False),
    )


BB = 64


def _attn_body(h_ref, it_ref, ii_ref, ti_ref, hs_ref, mf_ref, a1_ref, a2_ref,
               out_ref):
    hst = hs_ref[...]
    mf = mf_ref[...][:, :L]
    a1 = a1_ref[...][0:1, :].reshape(1, 1, DPAD)
    a2 = a2_ref[...][0:1, :].reshape(1, 1, DPAD)

    h = h_ref[...][:, :L, :]
    h = h * (ii_ref[...][:, :L] != 0).astype(jnp.float32)[:, :, None]
    itm = it_ref[...][:, :L, :]
    itm = itm * ((ti_ref[...][:, :L] != 0).astype(jnp.float32) * mf)[:, :, None]
    sess = jnp.sum(itm, axis=1) / jnp.sum(mf, axis=1)[:, None]

    att_n = jnp.sum(h * a1, axis=-1)
    att_n = jnp.where(att_n >= 0, att_n, ALPHA * att_n)
    w = jnp.where(hst > 0, hst * att_n[:, None, :], NEG)
    w = w - jnp.max(w, axis=2, keepdims=True)
    w = jnp.exp(w)
    w = w / jnp.sum(w, axis=2, keepdims=True)
    edge = lax.dot_general(w, h, (((2,), (1,)), ((0,), (0,))),
                           preferred_element_type=jnp.float32)

    att_e = jnp.sum(edge * a2, axis=-1)
    att_e = jnp.where(att_e >= 0, att_e, ALPHA * att_e)
    w2 = jnp.where(hst > 0, hst * att_e[:, :, None], NEG)
    w2 = w2 - jnp.max(w2, axis=1, keepdims=True)
    w2 = jnp.exp(w2)
    w2 = w2 / jnp.sum(w2, axis=1, keepdims=True)
    out = lax.dot_general(w2, edge, (((1,), (1,)), ((0,), (0,))),
                          preferred_element_type=jnp.float32)
    out = out + sess[:, None, :]
    out_ref[...] = out[:, :, :DIM]


def _attention(h, itm, ii, ti, Hs, maskf, a1b, a2b):
    grid = (B // BB,)
    return pl.pallas_call(
        _attn_body,
        grid=grid,
        in_specs=[
            pl.BlockSpec((BB, LP, DPAD), lambda i: (i, 0, 0)),
            pl.BlockSpec((BB, LP, DPAD), lambda i: (i, 0, 0)),
            pl.BlockSpec((BB, LP), lambda i: (i, 0)),
            pl.BlockSpec((BB, LP), lambda i: (i, 0)),
            pl.BlockSpec((BB, L, L), lambda i: (i, 0, 0)),
            pl.BlockSpec((BB, LP), lambda i: (i, 0)),
            pl.BlockSpec((8, DPAD), lambda i: (0, 0)),
            pl.BlockSpec((8, DPAD), lambda i: (0, 0)),
        ],
        out_specs=pl.BlockSpec((BB, L, DIM), lambda i: (i, 0, 0)),
        out_shape=jax.ShapeDtypeStruct((B, L, DIM), jnp.float32),
    )(h, itm, ii, ti, Hs, maskf, a1b, a2b)


def kernel(inputs, Hs, mask_item, item, embedding, a1, a2):
    inputs = inputs.astype(jnp.int32)
    item = item.astype(jnp.int32)
    fb = (lax.broadcasted_iota(jnp.int32, (B, LP - L), 0) * 131
          + lax.broadcasted_iota(jnp.int32, (B, LP - L), 1) * 17) % NROW
    ii56 = jnp.concatenate([inputs, fb], axis=1)
    ti56 = jnp.concatenate([item, (fb + 7) % NROW], axis=1)
    idx = jnp.concatenate([ii56.reshape(-1), ti56.reshape(-1)])
    safe = jnp.maximum(idx - 1, 0)

    table, embp_t = _build(embedding.T)
    h_rows, item_rows = _sc_gather()(table, safe)
    h3 = h_rows.reshape(B, LP, DPAD)
    it3 = item_rows.reshape(B, LP, DPAD)

    mf56 = jnp.pad(mask_item.astype(jnp.float32), ((0, 0), (0, LP - L)))
    a1b = jnp.pad(a1.reshape(1, DIM), ((0, 7), (0, DPAD - DIM)))
    a2b = jnp.pad(a2.reshape(1, DIM), ((0, 7), (0, DPAD - DIM)))
    h_local = _attention(h3, it3, ii56, ti56, jnp.swapaxes(Hs, 1, 2),
                         mf56, a1b, a2b)

    item_embeddings = embp_t.T
    return (h_local, item_embeddings)


# speedup vs baseline: 3.1754x; 1.0001x over previous
import functools

import jax
import jax.numpy as jnp
from jax import lax
from jax.experimental import pallas as pl
from jax.experimental.pallas import tpu as pltpu
from jax.experimental.pallas import tpu_sc as plsc

DIM = 64
DPAD = 128
NROW = 1000000
B = 1024
L = 50
LP = 56
ALPHA = 0.2
NEG = -1e9

NC = 2
NS = 16
NW = NC * NS
NIDX = 2 * B * LP
BPW = NIDX // NW
CH = 896
HALF = B * LP

CT = 8192


def _build_body(x_ref, pt_ref, tab_ref, emb_ref):
    i = pl.program_id(0)
    x = x_ref[...]
    eyep = (lax.broadcasted_iota(jnp.int32, (DIM, DPAD), 0)
            == lax.broadcasted_iota(jnp.int32, (DIM, DPAD), 1)
            ).astype(jnp.float32)
    tab_ref[...] = lax.dot_general(x, eyep, (((0,), (0,)), ((), ())),
                                   preferred_element_type=jnp.float32)
    pt = pt_ref[...]
    first = jnp.where(i == 0, 0.0, 1.0) * pt[:, DPAD - 1:DPAD]
    emb_ref[...] = jnp.concatenate([first, x[:, :CT - 1]], axis=1)


def _build(embT):
    return pl.pallas_call(
        _build_body,
        grid=(pl.cdiv(NROW + 1, CT),),
        in_specs=[
            pl.BlockSpec((DIM, CT), lambda i: (0, i)),
            pl.BlockSpec((DIM, DPAD),
                         lambda i: (0, jnp.maximum(i * (CT // DPAD) - 1, 0))),
        ],
        out_specs=[
            pl.BlockSpec((CT, DPAD), lambda i: (i, 0)),
            pl.BlockSpec((DIM, CT), lambda i: (0, i)),
        ],
        out_shape=(
            jax.ShapeDtypeStruct((NROW, DPAD), jnp.float32),
            jax.ShapeDtypeStruct((DIM, NROW + 1), jnp.float32),
        ),
    )(embT, embT)


def _gather_body(table, idx, out_h, out_i, idx_v, rows_v, sem):
    wid = lax.axis_index("s") * NC + lax.axis_index("c")
    base = wid * BPW
    pltpu.sync_copy(idx.at[pl.ds(base, BPW)], idx_v)
    for c in range(BPW // CH):
        pltpu.async_copy(table.at[idx_v.at[pl.ds(c * CH, CH)]], rows_v, sem).wait()
        off = base + c * CH

        @pl.when(wid < NW // 2)
        def _():
            pltpu.sync_copy(rows_v, out_h.at[pl.ds(off, CH)])

        @pl.when(wid >= NW // 2)
        def _():
            pltpu.sync_copy(rows_v, out_i.at[pl.ds(off - HALF, CH)])


@functools.cache
def _sc_gather():
    return pl.kernel(
        _gather_body,
        out_type=(
            jax.ShapeDtypeStruct((HALF, DPAD), jnp.float32),
            jax.ShapeDtypeStruct((HALF, DPAD), jnp.float32),
        ),
        mesh=plsc.VectorSubcoreMesh(
            core_axis_name="c", subcore_axis_name="s",
            num_cores=NC, num_subcores=NS,
        ),
        scratch_types=[
            pltpu.VMEM((BPW,), jnp.int32),
            pltpu.VMEM((CH, DPAD), jnp.float32),
            pltpu.SemaphoreType.DMA,
        ],
        compiler_params=pltpu.CompilerParams(use_tc_tiling_on_sc=False),
    )


BB = 128


def _attn_body(h_ref, it_ref, ii_ref, ti_ref, hs_ref, mf_ref, a1_ref, a2_ref,
               out_ref):
    hst = hs_ref[...]
    mf = mf_ref[...][:, :L]
    a1 = a1_ref[...][0:1, :].reshape(1, 1, DPAD)
    a2 = a2_ref[...][0:1, :].reshape(1, 1, DPAD)

    h = h_ref[...][:, :L, :]
    h = h * (ii_ref[...][:, :L] != 0).astype(jnp.float32)[:, :, None]
    itm = it_ref[...][:, :L, :]
    itm = itm * ((ti_ref[...][:, :L] != 0).astype(jnp.float32) * mf)[:, :, None]
    sess = jnp.sum(itm, axis=1) / jnp.sum(mf, axis=1)[:, None]

    att_n = jnp.sum(h * a1, axis=-1)
    att_n = jnp.where(att_n >= 0, att_n, ALPHA * att_n)
    w = jnp.where(hst > 0, hst * att_n[:, None, :], NEG)
    w = w - jnp.max(w, axis=2, keepdims=True)
    w = jnp.exp(w)
    w = w / jnp.sum(w, axis=2, keepdims=True)
    edge = lax.dot_general(w, h, (((2,), (1,)), ((0,), (0,))),
                           preferred_element_type=jnp.float32)

    att_e = jnp.sum(edge * a2, axis=-1)
    att_e = jnp.where(att_e >= 0, att_e, ALPHA * att_e)
    w2 = jnp.where(hst > 0, hst * att_e[:, :, None], NEG)
    w2 = w2 - jnp.max(w2, axis=1, keepdims=True)
    w2 = jnp.exp(w2)
    w2 = w2 / jnp.sum(w2, axis=1, keepdims=True)
    out = lax.dot_general(w2, edge, (((1,), (1,)), ((0,), (0,))),
                          preferred_element_type=jnp.float32)
    out = out + sess[:, None, :]
    out_ref[...] = out[:, :, :DIM]


def _attention(h, itm, ii, ti, Hs, maskf, a1b, a2b):
    grid = (B // BB,)
    return pl.pallas_call(
        _attn_body,
        grid=grid,
        in_specs=[
            pl.BlockSpec((BB, LP, DPAD), lambda i: (i, 0, 0)),
            pl.BlockSpec((BB, LP, DPAD), lambda i: (i, 0, 0)),
            pl.BlockSpec((BB, LP), lambda i: (i, 0)),
            pl.BlockSpec((BB, LP), lambda i: (i, 0)),
            pl.BlockSpec((BB, L, L), lambda i: (i, 0, 0)),
            pl.BlockSpec((BB, LP), lambda i: (i, 0)),
            pl.BlockSpec((8, DPAD), lambda i: (0, 0)),
            pl.BlockSpec((8, DPAD), lambda i: (0, 0)),
        ],
        out_specs=pl.BlockSpec((BB, L, DIM), lambda i: (i, 0, 0)),
        out_shape=jax.ShapeDtypeStruct((B, L, DIM), jnp.float32),
    )(h, itm, ii, ti, Hs, maskf, a1b, a2b)


def kernel(inputs, Hs, mask_item, item, embedding, a1, a2):
    inputs = inputs.astype(jnp.int32)
    item = item.astype(jnp.int32)
    fb = (lax.broadcasted_iota(jnp.int32, (B, LP - L), 0) * 131
          + lax.broadcasted_iota(jnp.int32, (B, LP - L), 1) * 17) % NROW
    ii56 = jnp.concatenate([inputs, fb], axis=1)
    ti56 = jnp.concatenate([item, (fb + 7) % NROW], axis=1)
    idx = jnp.concatenate([ii56.reshape(-1), ti56.reshape(-1)])
    safe = jnp.maximum(idx - 1, 0)

    table, embp_t = _build(embedding.T)
    h_rows, item_rows = _sc_gather()(table, safe)
    h3 = h_rows.reshape(B, LP, DPAD)
    it3 = item_rows.reshape(B, LP, DPAD)

    mf56 = jnp.pad(mask_item.astype(jnp.float32), ((0, 0), (0, LP - L)))
    a1b = jnp.pad(a1.reshape(1, DIM), ((0, 7), (0, DPAD - DIM)))
    a2b = jnp.pad(a2.reshape(1, DIM), ((0, 7), (0, DPAD - DIM)))
    h_local = _attention(h3, it3, ii56, ti56, jnp.swapaxes(Hs, 1, 2),
                         mf56, a1b, a2b)

    item_embeddings = embp_t.T
    return (h_local, item_embeddings)
